# baseline (speedup 1.0000x reference)
import functools

import jax
import jax.numpy as jnp
from jax import lax
from jax.experimental import pallas as pl
from jax.experimental.pallas import tpu as pltpu
from jax.experimental.pallas import tpu_sc as plsc

VOCAB = 100000
DIM = 64
B = 16384
N_CTX = 50
N_NEG = 20

L = 16
NC, NS = 2, 16
NW = NC * NS
B_PER_W = B // NW
C = 16
N_CHUNK = B_PER_W // C

_LN2 = 0.6931471805599453


def _softplus16(x):
    z = 1.0 + jnp.exp(x)
    zi = plsc.bitcast(z, jnp.int32)
    e = (zi >> 23) - 127
    m = plsc.bitcast((zi & 0x007FFFFF) | 0x3F800000, jnp.float32)
    t = m - 1.0
    s = t / (t + 2.0)
    s2 = s * s
    y0 = _LN2 * e.astype(jnp.float32) + 2.0 * s * (1.0 + s2 * (1.0 / 3.0 + s2 * 0.2))
    return y0 + z * jnp.exp(-y0) - 1.0


def _sc_body(ctx_idx_hbm, neg_idx_hbm, tgt_idx_hbm, ctx_tab_hbm, out_tab_hbm,
             out_hbm,
             ctx_idx_v, neg_idx_v, tgt_idx_v, ctx_rows0, ctx_rows1,
             neg_rows_v, tgt_rows_v, bsum_v, ctxsum_v, loss_v,
             sem_idx, sem_c0, sem_c1, sem_out):
    c = lax.axis_index("c")
    s = lax.axis_index("s")
    wid = s * NC + c
    base = wid * B_PER_W
    ctx_rows = (ctx_rows0, ctx_rows1)
    sem_ctx = (sem_c0, sem_c1)

    iota = lax.iota(jnp.int32, L)
    row_ctx = iota * N_CTX
    row_neg = iota * N_NEG

    def idx_copies(slot, g):
        b0 = base + g * C
        return (
            pltpu.make_async_copy(
                ctx_idx_hbm.at[pl.ds(b0 * N_CTX, C * N_CTX)],
                ctx_idx_v.at[slot], sem_idx),
            pltpu.make_async_copy(
                neg_idx_hbm.at[pl.ds(b0 * N_NEG, C * N_NEG)],
                neg_idx_v.at[slot], sem_idx),
            pltpu.make_async_copy(
                tgt_idx_hbm.at[pl.ds(b0, C)], tgt_idx_v.at[slot], sem_idx),
        )

    def ctx_copy(slot):
        return pltpu.make_async_copy(
            ctx_tab_hbm.at[ctx_idx_v.at[slot]], ctx_rows[slot], sem_ctx[slot])

    def out_copies(slot):
        return (
            pltpu.make_async_copy(
                out_tab_hbm.at[neg_idx_v.at[slot]], neg_rows_v, sem_out),
            pltpu.make_async_copy(
                out_tab_hbm.at[tgt_idx_v.at[slot]], tgt_rows_v, sem_out),
        )

    for g in (0, 1):
        for cp in idx_copies(g, g):
            cp.start()
    for cp in idx_copies(0, 0):
        cp.wait()
    ctx_copy(0).start()

    def loop_body(g2, loss_acc):
        for p in (0, 1):
            g = g2 * 2 + p
            q = 1 - p

            ctx_copy(p).wait()

            for cp in out_copies(p):
                cp.start()

            @pl.when(g + 1 < N_CHUNK)
            def _():
                for cp in idx_copies(q, g + 1):
                    cp.wait()
                ctx_copy(q).start()

            crows = ctx_rows[p]
            for b4 in range(C // 4):
                def r_body(r, accs):
                    out = []
                    for bi in range(4):
                        base_row = (b4 * 4 + bi) * N_CTX
                        for k in range(4):
                            out.append(accs[bi * 4 + k] +
                                       crows[base_row + r, pl.ds(k * L, L)])
                    return tuple(out)
                accs = lax.fori_loop(
                    0, N_CTX, r_body,
                    tuple(jnp.zeros((L,), jnp.float32) for _ in range(16)))
                for bi in range(4):
                    for k in range(4):
                        bsum_v[b4 * 4 + bi, pl.ds(k * L, L)] = accs[bi * 4 + k]

            def t_body(d, carry):
                ctxsum_v[d] = plsc.load_gather(
                    bsum_v, [iota, jnp.full((L,), d, jnp.int32)])
                return carry
            lax.fori_loop(0, DIM, t_body, 0)

            @pl.when(g + 2 < N_CHUNK)
            def _():
                for cp in idx_copies(p, g + 2):
                    cp.start()

            for cp in out_copies(p):
                cp.wait()

            DC = 4

            def make_dc_body(j_lo, j_hi, with_tgt):
                def dc_body(dc, scores):
                    d0 = dc * DC
                    ct = [ctxsum_v[d0 + k] for k in range(DC)]
                    cols = [jnp.full((L,), d0 + k, jnp.int32)
                            for k in range(DC)]
                    out = []
                    si = 0
                    if with_tgt:
                        s = scores[0]
                        for k in range(DC):
                            s = s + ct[k] * plsc.load_gather(
                                tgt_rows_v, [iota, cols[k]])
                        out.append(s)
                        si = 1
                    for j in range(j_lo, j_hi):
                        s = scores[si + j - j_lo]
                        rj = row_neg + j
                        for k in range(DC):
                            s = s + ct[k] * plsc.load_gather(
                                neg_rows_v, [rj, cols[k]])
                        out.append(s)
                    return tuple(out)
                return dc_body

            n1 = 1 + N_NEG // 2
            scores1 = lax.fori_loop(
                0, DIM // DC, make_dc_body(0, N_NEG // 2, True),
                tuple(jnp.zeros((L,), jnp.float32) for _ in range(n1)))
            n2 = N_NEG - N_NEG // 2
            scores2 = lax.fori_loop(
                0, DIM // DC, make_dc_body(N_NEG // 2, N_NEG, False),
                tuple(jnp.zeros((L,), jnp.float32) for _ in range(n2)))

            s_t = jnp.clip(scores1[0], -10.0, 10.0)
            loss_acc = loss_acc + _softplus16(-s_t)
            for s_n in list(scores1[1:]) + list(scores2):
                s_n = jnp.clip(s_n, -10.0, 10.0)
                loss_acc = loss_acc + _softplus16(s_n)
        return loss_acc

    loss_acc = lax.fori_loop(0, N_CHUNK // 2, loop_body,
                             jnp.zeros((L,), jnp.float32))
    loss_v[0] = loss_acc
    pltpu.sync_copy(loss_v, out_hbm.at[pl.ds(wid, 1)])


@jax.jit
def _sc_loss_partials(ctx_idx, neg_idx, tgt_idx, ctx_tab, out_tab):
    mesh = plsc.VectorSubcoreMesh(core_axis_name="c", subcore_axis_name="s")
    return pl.kernel(
        _sc_body,
        out_type=jax.ShapeDtypeStruct((NW, L), jnp.float32),
        mesh=mesh,
        compiler_params=pltpu.CompilerParams(use_tc_tiling_on_sc=False,
                                             needs_layout_passes=False),
        scratch_types=[
            pltpu.VMEM((2, C * N_CTX), jnp.int32),
            pltpu.VMEM((2, C * N_NEG), jnp.int32),
            pltpu.VMEM((2, C), jnp.int32),
            pltpu.VMEM((C * N_CTX, DIM), jnp.float32),
            pltpu.VMEM((C * N_CTX, DIM), jnp.float32),
            pltpu.VMEM((C * N_NEG, DIM), jnp.float32),
            pltpu.VMEM((C, DIM), jnp.float32),
            pltpu.VMEM((C, DIM), jnp.float32),
            pltpu.VMEM((DIM, L), jnp.float32),
            pltpu.VMEM((1, L), jnp.float32),
            pltpu.SemaphoreType.DMA,
            pltpu.SemaphoreType.DMA,
            pltpu.SemaphoreType.DMA,
            pltpu.SemaphoreType.DMA,
        ],
    )(ctx_idx, neg_idx, tgt_idx, ctx_tab, out_tab)


def _tc_finish_body(p_ref, o_ref):
    o_ref[0, 0] = jnp.sum(p_ref[...]) * (1.0 / B)


@jax.jit
def _tc_finish(partials):
    out = pl.pallas_call(
        _tc_finish_body,
        in_specs=[pl.BlockSpec((NW, L), lambda: (0, 0))],
        out_specs=pl.BlockSpec(memory_space=pltpu.SMEM),
        out_shape=jax.ShapeDtypeStruct((1, 1), jnp.float32),
    )(partials)
    return out[0, 0]


def kernel(pos_target, pos_contexts, pos_negatives, context_table, output_table):
    ctx_idx = pos_contexts.astype(jnp.int32).reshape(B * N_CTX)
    neg_idx = pos_negatives.astype(jnp.int32).reshape(B * N_NEG)
    tgt_idx = pos_target.astype(jnp.int32)
    partials = _sc_loss_partials(ctx_idx, neg_idx, tgt_idx,
                                 context_table, output_table)
    return _tc_finish(partials)
